# baseline (speedup 1.0000x reference)
import functools

import jax
import jax.numpy as jnp
from jax import lax
from jax.experimental import pallas as pl
from jax.experimental.pallas import tpu as pltpu
from jax.experimental.pallas import tpu_sc as plsc

EMBED = 256
NCODES = 8192
ROWS = 8 * 576
_SPLIT = 2560

_SC_CORES = 2
_SC_SUBCORES = 16
_SC_WORKERS = _SC_CORES * _SC_SUBCORES

_TM = 512
_EPS = 1e-07


def _normalize(x, axis):
    n = jnp.sqrt(jnp.sum(x * x, axis=axis, keepdims=True))
    return x / jnp.maximum(n, _EPS)


def _distance_body(z_ref, cb_ref, idx_ref, et_ref, ne_ref):
    i = pl.program_id(0)

    @pl.when(i == 0)
    def _init():
        cbt = cb_ref[...].T
        et = _normalize(cbt, axis=0)
        et_ref[...] = et.astype(jnp.bfloat16)
        ne_ref[...] = jnp.sum(et * et, axis=0, keepdims=True)

    zt = z_ref[...]
    zn = _normalize(zt, axis=1)
    sumz = jnp.sum(zn * zn, axis=1, keepdims=True)
    lhs = (2.0 * zn).astype(jnp.bfloat16)
    s2 = jnp.dot(lhs, et_ref[...],
                 preferred_element_type=jnp.float32)
    d = (sumz - s2) + ne_ref[...]
    idx = jnp.argmin(d, axis=1).astype(jnp.int32)
    idx_ref[0, 0, :] = idx


def _distance_indices(z_flat, codebook, row0, nrows):
    grid = nrows // _TM
    t0 = row0 // _TM
    return pl.pallas_call(
        _distance_body,
        grid=(grid,),
        in_specs=[
            pl.BlockSpec((_TM, EMBED), lambda i: (i + t0, 0)),
            pl.BlockSpec((NCODES, EMBED), lambda i: (0, 0)),
        ],
        out_specs=pl.BlockSpec((1, 1, _TM), lambda i: (i, 0, 0)),
        out_shape=jax.ShapeDtypeStruct((grid, 1, _TM), jnp.int32),
        scratch_shapes=[
            pltpu.VMEM((EMBED, NCODES), jnp.bfloat16),
            pltpu.VMEM((1, NCODES), jnp.float32),
        ],
    )(z_flat, codebook)


def _make_sc_gather_body(rows_per_worker):
    def body(cb_hbm, idx_hbm, out_hbm, idx_v, rows_v, sem):
        wid = lax.axis_index("s") * _SC_CORES + lax.axis_index("c")
        base = wid * rows_per_worker
        pltpu.sync_copy(idx_hbm.at[pl.ds(base, rows_per_worker)], idx_v)
        pltpu.async_copy(cb_hbm.at[idx_v], rows_v, sem).wait()
        pltpu.sync_copy(rows_v, out_hbm.at[pl.ds(base, rows_per_worker)])
    return body


def _sc_gather(codebook, idx_flat, nrows):
    rpw = nrows // _SC_WORKERS
    mesh = plsc.VectorSubcoreMesh(core_axis_name="c", subcore_axis_name="s")
    run = functools.partial(
        pl.kernel,
        mesh=mesh,
        out_type=jax.ShapeDtypeStruct((nrows, EMBED), jnp.float32),
        scratch_types=[
            pltpu.VMEM((rpw,), jnp.int32),
            pltpu.VMEM((rpw, EMBED), jnp.float32),
            pltpu.SemaphoreType.DMA,
        ],
    )(_make_sc_gather_body(rpw))
    return run(codebook, idx_flat)


def _output_body(z_ref, zq_ref, out_ref, loss_ref):
    i = pl.program_id(0)
    zt = z_ref[...]
    zn = _normalize(zt, axis=1)
    qn = _normalize(zq_ref[...], axis=1)
    out_ref[...] = zt + (qn - zt)
    diff = zn - qn
    part = jnp.sum(diff * diff)

    @pl.when(i == 0)
    def _init():
        loss_ref[...] = jnp.zeros_like(loss_ref)

    loss_ref[...] = loss_ref[...] + part


def _output_and_loss(z_flat, zq_raw, row0, nrows, tm):
    t0 = row0 // tm
    return pl.pallas_call(
        _output_body,
        grid=(nrows // tm,),
        in_specs=[
            pl.BlockSpec((tm, EMBED), lambda i: (i + t0, 0)),
            pl.BlockSpec((tm, EMBED), lambda i: (i, 0)),
        ],
        out_specs=[
            pl.BlockSpec((tm, EMBED), lambda i: (i, 0)),
            pl.BlockSpec((1, 1), lambda i: (0, 0)),
        ],
        out_shape=[
            jax.ShapeDtypeStruct((nrows, EMBED), jnp.float32),
            jax.ShapeDtypeStruct((1, 1), jnp.float32),
        ],
    )(z_flat, zq_raw)


def kernel(z, codebook):
    z_flat = z.reshape(ROWS, EMBED)
    na, nb = _SPLIT, ROWS - _SPLIT
    idx_a = _distance_indices(z_flat, codebook, 0, na).reshape(na)
    idx_b = _distance_indices(z_flat, codebook, na, nb).reshape(nb)
    zq_a = _sc_gather(codebook, idx_a, na)
    zq_b = _sc_gather(codebook, idx_b, nb)
    out_a, loss_a = _output_and_loss(z_flat, zq_a, 0, na, 1280)
    out_b, loss_b = _output_and_loss(z_flat, zq_b, na, nb, 512)
    z_q_st = jnp.concatenate([out_a, out_b], axis=0).reshape(z.shape)
    loss = ((loss_a + loss_b) * (1.25 / (ROWS * EMBED))).reshape(())
    idx = jnp.concatenate([idx_a, idx_b], axis=0).reshape(z.shape[:-1])
    return (z_q_st, loss, idx)


# speedup vs baseline: 1.1725x; 1.1725x over previous
import functools

import jax
import jax.numpy as jnp
from jax import lax
from jax.experimental import pallas as pl
from jax.experimental.pallas import tpu as pltpu
from jax.experimental.pallas import tpu_sc as plsc

EMBED = 256
NCODES = 8192
ROWS = 8 * 576

_SC_CORES = 2
_SC_SUBCORES = 16
_SC_WORKERS = _SC_CORES * _SC_SUBCORES
_ROWS_PER_WORKER = ROWS // _SC_WORKERS

_TM = 512
_GRID = ROWS // _TM
_TM2 = 1152
_GRID2 = ROWS // _TM2
_EPS = 1e-07


def _normalize(x, axis):
    n = jnp.sqrt(jnp.sum(x * x, axis=axis, keepdims=True))
    return x / jnp.maximum(n, _EPS)


def _distance_body(z_ref, cb_ref, idx_ref, et_ref, ne_ref):
    i = pl.program_id(0)

    @pl.when(i == 0)
    def _init():
        cbt = cb_ref[...].T
        et = _normalize(cbt, axis=0)
        et_ref[...] = et.astype(jnp.bfloat16)
        ne_ref[...] = jnp.sum(et * et, axis=0, keepdims=True)

    zt = z_ref[...]
    zn = _normalize(zt, axis=1)
    sumz = jnp.sum(zn * zn, axis=1, keepdims=True)
    lhs = (2.0 * zn).astype(jnp.bfloat16)
    s2 = jnp.dot(lhs, et_ref[...],
                 preferred_element_type=jnp.float32)
    d = (sumz - s2) + ne_ref[...]
    idx = jnp.argmin(d, axis=1).astype(jnp.int32)
    idx_ref[0, 0, :] = idx


def _distance_indices(z_flat, codebook):
    return pl.pallas_call(
        _distance_body,
        grid=(_GRID,),
        in_specs=[
            pl.BlockSpec((_TM, EMBED), lambda i: (i, 0)),
            pl.BlockSpec((NCODES, EMBED), lambda i: (0, 0)),
        ],
        out_specs=pl.BlockSpec((1, 1, _TM), lambda i: (i, 0, 0)),
        out_shape=jax.ShapeDtypeStruct((_GRID, 1, _TM), jnp.int32),
        scratch_shapes=[
            pltpu.VMEM((EMBED, NCODES), jnp.bfloat16),
            pltpu.VMEM((1, NCODES), jnp.float32),
        ],
    )(z_flat, codebook)


def _sc_gather_body(cb_hbm, idx_hbm, out_hbm, idx_v, rows_v, sem):
    wid = lax.axis_index("s") * _SC_CORES + lax.axis_index("c")
    base = wid * _ROWS_PER_WORKER
    pltpu.sync_copy(idx_hbm.at[pl.ds(base, _ROWS_PER_WORKER)], idx_v)
    pltpu.async_copy(cb_hbm.at[idx_v], rows_v, sem).wait()
    pltpu.sync_copy(rows_v, out_hbm.at[pl.ds(base, _ROWS_PER_WORKER)])


def _sc_gather(codebook, idx_flat):
    mesh = plsc.VectorSubcoreMesh(core_axis_name="c", subcore_axis_name="s")
    run = functools.partial(
        pl.kernel,
        mesh=mesh,
        out_type=jax.ShapeDtypeStruct((ROWS, EMBED), jnp.float32),
        scratch_types=[
            pltpu.VMEM((_ROWS_PER_WORKER,), jnp.int32),
            pltpu.VMEM((_ROWS_PER_WORKER, EMBED), jnp.float32),
            pltpu.SemaphoreType.DMA,
        ],
    )(_sc_gather_body)
    return run(codebook, idx_flat)


def _output_body(z_ref, zq_ref, out_ref, loss_ref):
    i = pl.program_id(0)
    zt = z_ref[...]
    zn = _normalize(zt, axis=1)
    qn = _normalize(zq_ref[...], axis=1)
    out_ref[...] = zt + (qn - zt)
    diff = zn - qn
    part = jnp.sum(diff * diff)

    @pl.when(i == 0)
    def _init():
        loss_ref[...] = jnp.zeros_like(loss_ref)

    loss_ref[...] = loss_ref[...] + part

    @pl.when(i == _GRID2 - 1)
    def _fin():
        loss_ref[...] = loss_ref[...] * (1.25 / (ROWS * EMBED))


def _output_and_loss(z_flat, zq_raw):
    return pl.pallas_call(
        _output_body,
        grid=(_GRID2,),
        in_specs=[
            pl.BlockSpec((_TM2, EMBED), lambda i: (i, 0)),
            pl.BlockSpec((_TM2, EMBED), lambda i: (i, 0)),
        ],
        out_specs=[
            pl.BlockSpec((_TM2, EMBED), lambda i: (i, 0)),
            pl.BlockSpec((1, 1), lambda i: (0, 0)),
        ],
        out_shape=[
            jax.ShapeDtypeStruct((ROWS, EMBED), jnp.float32),
            jax.ShapeDtypeStruct((1, 1), jnp.float32),
        ],
    )(z_flat, zq_raw)


def kernel(z, codebook):
    z_flat = z.reshape(ROWS, EMBED)
    idx = _distance_indices(z_flat, codebook).reshape(ROWS)
    zq_raw = _sc_gather(codebook, idx)
    z_q_st, loss = _output_and_loss(z_flat, zq_raw)
    return (z_q_st.reshape(z.shape), loss.reshape(()),
            idx.reshape(z.shape[:-1]))


# speedup vs baseline: 1.1855x; 1.0111x over previous
import functools

import jax
import jax.numpy as jnp
from jax import lax
from jax.experimental import pallas as pl
from jax.experimental.pallas import tpu as pltpu
from jax.experimental.pallas import tpu_sc as plsc

EMBED = 256
NCODES = 8192
ROWS = 8 * 576

_SC_CORES = 2
_SC_SUBCORES = 16
_SC_WORKERS = _SC_CORES * _SC_SUBCORES
_ROWS_PER_WORKER = ROWS // _SC_WORKERS

_TM = 576
_GRID = ROWS // _TM
_TM2 = 1152
_GRID2 = ROWS // _TM2
_EPS = 1e-07


def _normalize(x, axis):
    n = jnp.sqrt(jnp.sum(x * x, axis=axis, keepdims=True))
    return x / jnp.maximum(n, _EPS)


def _distance_body(z_ref, cb_ref, idx_ref, et_ref, ne_ref):
    i = pl.program_id(0)

    @pl.when(i == 0)
    def _init():
        cbt = cb_ref[...].T
        et = _normalize(cbt, axis=0)
        et_ref[...] = et.astype(jnp.bfloat16)
        ne_ref[...] = jnp.sum(et * et, axis=0, keepdims=True)

    zt = z_ref[...]
    zn = _normalize(zt, axis=1)
    sumz = jnp.sum(zn * zn, axis=1, keepdims=True)
    lhs = (2.0 * zn).astype(jnp.bfloat16)
    s2 = jnp.dot(lhs, et_ref[...],
                 preferred_element_type=jnp.float32)
    d = (sumz - s2) + ne_ref[...]
    idx = jnp.argmin(d, axis=1).astype(jnp.int32)
    idx_ref[0, 0, :] = idx


def _distance_indices(z_flat, codebook):
    return pl.pallas_call(
        _distance_body,
        grid=(_GRID,),
        in_specs=[
            pl.BlockSpec((_TM, EMBED), lambda i: (i, 0)),
            pl.BlockSpec((NCODES, EMBED), lambda i: (0, 0)),
        ],
        out_specs=pl.BlockSpec((1, 1, _TM), lambda i: (i, 0, 0)),
        out_shape=jax.ShapeDtypeStruct((_GRID, 1, _TM), jnp.int32),
        scratch_shapes=[
            pltpu.VMEM((EMBED, NCODES), jnp.bfloat16),
            pltpu.VMEM((1, NCODES), jnp.float32),
        ],
    )(z_flat, codebook)


def _sc_gather_body(cb_hbm, idx_hbm, out_hbm, idx_v, rows_v, sem):
    wid = lax.axis_index("s") * _SC_CORES + lax.axis_index("c")
    base = wid * _ROWS_PER_WORKER
    pltpu.sync_copy(idx_hbm.at[pl.ds(base, _ROWS_PER_WORKER)], idx_v)
    pltpu.async_copy(cb_hbm.at[idx_v], rows_v, sem).wait()
    pltpu.sync_copy(rows_v, out_hbm.at[pl.ds(base, _ROWS_PER_WORKER)])


def _sc_gather(codebook, idx_flat):
    mesh = plsc.VectorSubcoreMesh(core_axis_name="c", subcore_axis_name="s")
    run = functools.partial(
        pl.kernel,
        mesh=mesh,
        out_type=jax.ShapeDtypeStruct((ROWS, EMBED), jnp.float32),
        scratch_types=[
            pltpu.VMEM((_ROWS_PER_WORKER,), jnp.int32),
            pltpu.VMEM((_ROWS_PER_WORKER, EMBED), jnp.float32),
            pltpu.SemaphoreType.DMA,
        ],
    )(_sc_gather_body)
    return run(codebook, idx_flat)


def _output_body(z_ref, zq_ref, out_ref, loss_ref):
    i = pl.program_id(0)
    zt = z_ref[...]
    zn = _normalize(zt, axis=1)
    qn = _normalize(zq_ref[...], axis=1)
    out_ref[...] = zt + (qn - zt)
    diff = zn - qn
    part = jnp.sum(diff * diff)

    @pl.when(i == 0)
    def _init():
        loss_ref[...] = jnp.zeros_like(loss_ref)

    loss_ref[...] = loss_ref[...] + part

    @pl.when(i == _GRID2 - 1)
    def _fin():
        loss_ref[...] = loss_ref[...] * (1.25 / (ROWS * EMBED))


def _output_and_loss(z_flat, zq_raw):
    return pl.pallas_call(
        _output_body,
        grid=(_GRID2,),
        in_specs=[
            pl.BlockSpec((_TM2, EMBED), lambda i: (i, 0)),
            pl.BlockSpec((_TM2, EMBED), lambda i: (i, 0)),
        ],
        out_specs=[
            pl.BlockSpec((_TM2, EMBED), lambda i: (i, 0)),
            pl.BlockSpec((1, 1), lambda i: (0, 0)),
        ],
        out_shape=[
            jax.ShapeDtypeStruct((ROWS, EMBED), jnp.float32),
            jax.ShapeDtypeStruct((1, 1), jnp.float32),
        ],
    )(z_flat, zq_raw)


def kernel(z, codebook):
    z_flat = z.reshape(ROWS, EMBED)
    idx = _distance_indices(z_flat, codebook).reshape(ROWS)
    zq_raw = _sc_gather(codebook, idx)
    z_q_st, loss = _output_and_loss(z_flat, zq_raw)
    return (z_q_st.reshape(z.shape), loss.reshape(()),
            idx.reshape(z.shape[:-1]))


# speedup vs baseline: 1.3592x; 1.1465x over previous
import functools

import jax
import jax.numpy as jnp
from jax import lax
from jax.experimental import pallas as pl
from jax.experimental.pallas import tpu as pltpu
from jax.experimental.pallas import tpu_sc as plsc

EMBED = 256
NCODES = 8192
ROWS = 8 * 576

_SC_CORES = 2
_SC_SUBCORES = 16
_SC_WORKERS = _SC_CORES * _SC_SUBCORES
_ROWS_PER_WORKER = ROWS // _SC_WORKERS

_TM = 576
_GRID = ROWS // _TM
_TM2 = 1152
_GRID2 = ROWS // _TM2
_EPS = 1e-07


def _normalize(x, axis):
    n = jnp.sqrt(jnp.sum(x * x, axis=axis, keepdims=True))
    return x / jnp.maximum(n, _EPS)


def _distance_body(z_ref, cb_ref, idx_ref, et_ref, ne_ref):
    i = pl.program_id(0)

    @pl.when(i == 0)
    def _init():
        cbt = cb_ref[...].T
        et = _normalize(cbt, axis=0)
        et_ref[...] = et.astype(jnp.bfloat16)
        ne_ref[...] = jnp.sum(et * et, axis=0, keepdims=True)

    zt = z_ref[...]
    zn = _normalize(zt, axis=1)
    lhs = (2.0 * zn).astype(jnp.bfloat16)
    s2 = jnp.dot(lhs, et_ref[...],
                 preferred_element_type=jnp.float32)
    d = ne_ref[...] - s2
    idx = jnp.argmin(d, axis=1).astype(jnp.int32)
    idx_ref[0, 0, :] = idx


def _distance_indices(z_flat, codebook):
    return pl.pallas_call(
        _distance_body,
        grid=(_GRID,),
        in_specs=[
            pl.BlockSpec((_TM, EMBED), lambda i: (i, 0)),
            pl.BlockSpec((NCODES, EMBED), lambda i: (0, 0)),
        ],
        out_specs=pl.BlockSpec((1, 1, _TM), lambda i: (i, 0, 0)),
        out_shape=jax.ShapeDtypeStruct((_GRID, 1, _TM), jnp.int32),
        scratch_shapes=[
            pltpu.VMEM((EMBED, NCODES), jnp.bfloat16),
            pltpu.VMEM((1, NCODES), jnp.float32),
        ],
    )(z_flat, codebook)


def _sc_gather_body(cb_hbm, idx_hbm, out_hbm, idx_v, rows_v, sem):
    wid = lax.axis_index("s") * _SC_CORES + lax.axis_index("c")
    base = wid * _ROWS_PER_WORKER
    pltpu.sync_copy(idx_hbm.at[pl.ds(base, _ROWS_PER_WORKER)], idx_v)
    pltpu.async_copy(cb_hbm.at[idx_v], rows_v, sem).wait()
    pltpu.sync_copy(rows_v, out_hbm.at[pl.ds(base, _ROWS_PER_WORKER)])


def _sc_gather(codebook, idx_flat):
    mesh = plsc.VectorSubcoreMesh(core_axis_name="c", subcore_axis_name="s")
    run = functools.partial(
        pl.kernel,
        mesh=mesh,
        out_type=jax.ShapeDtypeStruct((ROWS, EMBED), jnp.float32),
        scratch_types=[
            pltpu.VMEM((_ROWS_PER_WORKER,), jnp.int32),
            pltpu.VMEM((_ROWS_PER_WORKER, EMBED), jnp.float32),
            pltpu.SemaphoreType.DMA,
        ],
    )(_sc_gather_body)
    return run(codebook, idx_flat)


def _output_body(z_ref, zq_ref, out_ref, loss_ref):
    i = pl.program_id(0)
    zt = z_ref[...]
    zn = _normalize(zt, axis=1)
    qn = _normalize(zq_ref[...], axis=1)
    out_ref[...] = zt + (qn - zt)
    diff = zn - qn
    part = jnp.sum(diff * diff)

    @pl.when(i == 0)
    def _init():
        loss_ref[...] = jnp.zeros_like(loss_ref)

    loss_ref[...] = loss_ref[...] + part

    @pl.when(i == _GRID2 - 1)
    def _fin():
        loss_ref[...] = loss_ref[...] * (1.25 / (ROWS * EMBED))


def _output_and_loss(z_flat, zq_raw):
    return pl.pallas_call(
        _output_body,
        grid=(_GRID2,),
        in_specs=[
            pl.BlockSpec((_TM2, EMBED), lambda i: (i, 0)),
            pl.BlockSpec((_TM2, EMBED), lambda i: (i, 0)),
        ],
        out_specs=[
            pl.BlockSpec((_TM2, EMBED), lambda i: (i, 0)),
            pl.BlockSpec((1, 1), lambda i: (0, 0)),
        ],
        out_shape=[
            jax.ShapeDtypeStruct((ROWS, EMBED), jnp.float32),
            jax.ShapeDtypeStruct((1, 1), jnp.float32),
        ],
    )(z_flat, zq_raw)


def kernel(z, codebook):
    z_flat = z.reshape(ROWS, EMBED)
    idx = _distance_indices(z_flat, codebook).reshape(ROWS)
    zq_raw = _sc_gather(codebook, idx)
    z_q_st, loss = _output_and_loss(z_flat, zq_raw)
    return (z_q_st.reshape(z.shape), loss.reshape(()),
            idx.reshape(z.shape[:-1]))


# speedup vs baseline: 1.3678x; 1.0063x over previous
import functools

import jax
import jax.numpy as jnp
from jax import lax
from jax.experimental import pallas as pl
from jax.experimental.pallas import tpu as pltpu
from jax.experimental.pallas import tpu_sc as plsc

EMBED = 256
NCODES = 8192
ROWS = 8 * 576

_SC_CORES = 2
_SC_SUBCORES = 16
_SC_WORKERS = _SC_CORES * _SC_SUBCORES
_ROWS_PER_WORKER = ROWS // _SC_WORKERS

_TM = 768
_GRID = ROWS // _TM
_TM2 = 1152
_GRID2 = ROWS // _TM2
_EPS = 1e-07


def _normalize(x, axis):
    n = jnp.sqrt(jnp.sum(x * x, axis=axis, keepdims=True))
    return x / jnp.maximum(n, _EPS)


def _distance_body(z_ref, cb_ref, idx_ref, et_ref, ne_ref):
    i = pl.program_id(0)

    @pl.when(i == 0)
    def _init():
        cbt = cb_ref[...].T
        et = _normalize(cbt, axis=0)
        et_ref[...] = et.astype(jnp.bfloat16)
        ne_ref[...] = jnp.sum(et * et, axis=0, keepdims=True)

    zt = z_ref[...]
    zn = _normalize(zt, axis=1)
    lhs = (2.0 * zn).astype(jnp.bfloat16)
    s2 = jnp.dot(lhs, et_ref[...],
                 preferred_element_type=jnp.float32)
    d = ne_ref[...] - s2
    idx = jnp.argmin(d, axis=1).astype(jnp.int32)
    idx_ref[0, 0, :] = idx


def _distance_indices(z_flat, codebook):
    return pl.pallas_call(
        _distance_body,
        grid=(_GRID,),
        in_specs=[
            pl.BlockSpec((_TM, EMBED), lambda i: (i, 0)),
            pl.BlockSpec((NCODES, EMBED), lambda i: (0, 0)),
        ],
        out_specs=pl.BlockSpec((1, 1, _TM), lambda i: (i, 0, 0)),
        out_shape=jax.ShapeDtypeStruct((_GRID, 1, _TM), jnp.int32),
        scratch_shapes=[
            pltpu.VMEM((EMBED, NCODES), jnp.bfloat16),
            pltpu.VMEM((1, NCODES), jnp.float32),
        ],
    )(z_flat, codebook)


def _sc_gather_body(cb_hbm, idx_hbm, out_hbm, idx_v, rows_v, sem):
    wid = lax.axis_index("s") * _SC_CORES + lax.axis_index("c")
    base = wid * _ROWS_PER_WORKER
    pltpu.sync_copy(idx_hbm.at[pl.ds(base, _ROWS_PER_WORKER)], idx_v)
    pltpu.async_copy(cb_hbm.at[idx_v], rows_v, sem).wait()
    pltpu.sync_copy(rows_v, out_hbm.at[pl.ds(base, _ROWS_PER_WORKER)])


def _sc_gather(codebook, idx_flat):
    mesh = plsc.VectorSubcoreMesh(core_axis_name="c", subcore_axis_name="s")
    run = functools.partial(
        pl.kernel,
        mesh=mesh,
        out_type=jax.ShapeDtypeStruct((ROWS, EMBED), jnp.float32),
        scratch_types=[
            pltpu.VMEM((_ROWS_PER_WORKER,), jnp.int32),
            pltpu.VMEM((_ROWS_PER_WORKER, EMBED), jnp.float32),
            pltpu.SemaphoreType.DMA,
        ],
    )(_sc_gather_body)
    return run(codebook, idx_flat)


def _output_body(z_ref, zq_ref, out_ref, loss_ref):
    i = pl.program_id(0)
    zt = z_ref[...]
    zn = _normalize(zt, axis=1)
    qn = _normalize(zq_ref[...], axis=1)
    out_ref[...] = zt + (qn - zt)
    diff = zn - qn
    part = jnp.sum(diff * diff)

    @pl.when(i == 0)
    def _init():
        loss_ref[...] = jnp.zeros_like(loss_ref)

    loss_ref[...] = loss_ref[...] + part

    @pl.when(i == _GRID2 - 1)
    def _fin():
        loss_ref[...] = loss_ref[...] * (1.25 / (ROWS * EMBED))


def _output_and_loss(z_flat, zq_raw):
    return pl.pallas_call(
        _output_body,
        grid=(_GRID2,),
        in_specs=[
            pl.BlockSpec((_TM2, EMBED), lambda i: (i, 0)),
            pl.BlockSpec((_TM2, EMBED), lambda i: (i, 0)),
        ],
        out_specs=[
            pl.BlockSpec((_TM2, EMBED), lambda i: (i, 0)),
            pl.BlockSpec((1, 1), lambda i: (0, 0)),
        ],
        out_shape=[
            jax.ShapeDtypeStruct((ROWS, EMBED), jnp.float32),
            jax.ShapeDtypeStruct((1, 1), jnp.float32),
        ],
    )(z_flat, zq_raw)


def kernel(z, codebook):
    z_flat = z.reshape(ROWS, EMBED)
    idx = _distance_indices(z_flat, codebook).reshape(ROWS)
    zq_raw = _sc_gather(codebook, idx)
    z_q_st, loss = _output_and_loss(z_flat, zq_raw)
    return (z_q_st.reshape(z.shape), loss.reshape(()),
            idx.reshape(z.shape[:-1]))
